# baseline (speedup 1.0000x reference)
import functools

import jax
import jax.numpy as jnp
from jax import lax
from jax.experimental import pallas as pl
from jax.experimental.pallas import tpu as pltpu
from jax.experimental.pallas import tpu_sc as plsc

SEQ_LEN = 8192
NUM_CORES = 1
NUM_SUBCORES = 16
LANES = 16
NUM_WORKERS = NUM_CORES * NUM_SUBCORES
BPW = SEQ_LEN // NUM_WORKERS
NCHUNK = 4
CHUNK = BPW // NCHUNK

_mesh = plsc.VectorSubcoreMesh(core_axis_name="c", subcore_axis_name="s", num_cores=1)


@functools.partial(
    pl.kernel,
    mesh=_mesh,
    out_type=jax.ShapeDtypeStruct((SEQ_LEN,), jnp.float32),
    scratch_types=[
        pltpu.VMEM((BPW,), jnp.int32),
        pltpu.VMEM((BPW,), jnp.float32),
        pltpu.VMEM((BPW,), jnp.float32),
        pltpu.VMEM((BPW,), jnp.float32),
        pltpu.SemaphoreType.DMA,
        pltpu.SemaphoreType.DMA,
        pltpu.SemaphoreType.DMA,
        pltpu.SemaphoreType.DMA,
        pltpu.SemaphoreType.DMA,
        pltpu.SemaphoreType.DMA,
    ],
)
def _shmoof_sc(idx_hbm, kmer_hbm, site_hbm, out_hbm, idx_v, lk_v, ls_v, out_v,
               s_sem, g0_sem, g1_sem, g2_sem, g3_sem, out_sem):
    wid = lax.axis_index("s") * NUM_CORES + lax.axis_index("c")
    base = wid * BPW
    pltpu.sync_copy(idx_hbm.at[pl.ds(base, BPW)], idx_v)
    g_sems = [g0_sem, g1_sem, g2_sem, g3_sem]
    gathers = []
    for c in range(NCHUNK):
        lo = c * CHUNK
        gathers.append(pltpu.async_copy(
            kmer_hbm.at[idx_v.at[pl.ds(lo, CHUNK)]],
            lk_v.at[pl.ds(lo, CHUNK)], g_sems[c]))
    site = pltpu.async_copy(site_hbm.at[pl.ds(base, BPW)], ls_v, s_sem)
    site.wait()
    outs = []
    for c in range(NCHUNK):
        lo = c * CHUNK
        gathers[c].wait()
        for i in range(CHUNK // LANES):
            sl = pl.ds(lo + i * LANES, LANES)
            out_v[sl] = jnp.exp(lk_v[sl] + ls_v[sl])
        outs.append(pltpu.async_copy(
            out_v.at[pl.ds(lo, CHUNK)], out_hbm.at[pl.ds(base + lo, CHUNK)],
            out_sem))
    for o in outs:
        o.wait()


def kernel(encoded_parent, log_kmer_rates, log_site_rates):
    return _shmoof_sc(
        encoded_parent,
        log_kmer_rates.reshape(-1),
        log_site_rates.reshape(-1)[:SEQ_LEN],
    )


# speedup vs baseline: 1.0069x; 1.0069x over previous
import functools

import jax
import jax.numpy as jnp
from jax import lax
from jax.experimental import pallas as pl
from jax.experimental.pallas import tpu as pltpu
from jax.experimental.pallas import tpu_sc as plsc

SEQ_LEN = 8192
NUM_CORES = 1
NUM_SUBCORES = 16
LANES = 16
NUM_WORKERS = NUM_CORES * NUM_SUBCORES
BPW = SEQ_LEN // NUM_WORKERS

_mesh = plsc.VectorSubcoreMesh(core_axis_name="c", subcore_axis_name="s", num_cores=1)


@functools.partial(
    pl.kernel,
    mesh=_mesh,
    out_type=jax.ShapeDtypeStruct((SEQ_LEN,), jnp.float32),
    scratch_types=[
        pltpu.VMEM((BPW,), jnp.int32),
        pltpu.VMEM((BPW,), jnp.float32),
        pltpu.VMEM((BPW,), jnp.float32),
        pltpu.VMEM((BPW,), jnp.float32),
        pltpu.SemaphoreType.DMA,
        pltpu.SemaphoreType.DMA,
        pltpu.SemaphoreType.DMA,
    ],
)
def _shmoof_sc(idx_hbm, kmer_hbm, site_hbm, out_hbm, idx_v, lk_v, ls_v, out_v,
               g0_sem, g1_sem, out_sem):
    wid = lax.axis_index("s") * NUM_CORES + lax.axis_index("c")
    base = wid * BPW
    half = BPW // 2
    pltpu.sync_copy(idx_hbm.at[pl.ds(base, BPW)], idx_v)
    g0 = pltpu.async_copy(
        kmer_hbm.at[idx_v.at[pl.ds(0, half)]], lk_v.at[pl.ds(0, half)], g0_sem)
    g1 = pltpu.async_copy(
        kmer_hbm.at[idx_v.at[pl.ds(half, half)]], lk_v.at[pl.ds(half, half)], g1_sem)
    pltpu.sync_copy(site_hbm.at[pl.ds(base, BPW)], ls_v)
    g0.wait()
    for i in range(half // LANES):
        sl = pl.ds(i * LANES, LANES)
        out_v[sl] = jnp.exp(lk_v[sl] + ls_v[sl])
    o0 = pltpu.async_copy(
        out_v.at[pl.ds(0, half)], out_hbm.at[pl.ds(base, half)], out_sem)
    g1.wait()
    for i in range(half // LANES, BPW // LANES):
        sl = pl.ds(i * LANES, LANES)
        out_v[sl] = jnp.exp(lk_v[sl] + ls_v[sl])
    o1 = pltpu.async_copy(
        out_v.at[pl.ds(half, half)], out_hbm.at[pl.ds(base + half, half)], out_sem)
    o0.wait()
    o1.wait()


def kernel(encoded_parent, log_kmer_rates, log_site_rates):
    return _shmoof_sc(
        encoded_parent,
        log_kmer_rates.reshape(-1),
        log_site_rates.reshape(-1)[:SEQ_LEN],
    )
